# baseline (speedup 1.0000x reference)
import jax
import jax.numpy as jnp
from jax.experimental import pallas as pl

_BN = 16384


def _body(a_ref, b_ref, xui_ref, a_out_ref, b_out_ref):
    av = a_ref[...]
    bv = b_ref[...]
    a_out_ref[...] = av
    b_out_ref[...] = bv
    xui_ref[...] = jnp.sum(av * bv, axis=0)


def kernel(gum, gim):
    n_rows, n_cols = gum.shape
    a = gum.T
    b = gim.T
    grid = (n_rows // _BN,)
    xui, a_o, b_o = pl.pallas_call(
        _body,
        grid=grid,
        in_specs=[
            pl.BlockSpec((n_cols, _BN), lambda i: (0, i)),
            pl.BlockSpec((n_cols, _BN), lambda i: (0, i)),
        ],
        out_specs=[
            pl.BlockSpec((_BN,), lambda i: (i,)),
            pl.BlockSpec((n_cols, _BN), lambda i: (0, i)),
            pl.BlockSpec((n_cols, _BN), lambda i: (0, i)),
        ],
        out_shape=[
            jax.ShapeDtypeStruct((n_rows,), jnp.float32),
            jax.ShapeDtypeStruct((n_cols, n_rows), jnp.float32),
            jax.ShapeDtypeStruct((n_cols, n_rows), jnp.float32),
        ],
    )(a, b)
    return (xui, a_o.T, b_o.T)
